# baseline (speedup 1.0000x reference)
import functools

import jax
import jax.numpy as jnp
import numpy as np
from jax import lax
from jax.experimental import pallas as pl
from jax.experimental.pallas import tpu as pltpu
from jax.experimental.pallas import tpu_sc as plsc

L = 16
T = 2 ** 19
F = 2
BASE_RES = 16
PER_LEVEL_SCALE = 1.5
PRIMES = (1, 2654435761, 805459861, 3674653429)

B = 16384
NW = 32
PPW = B // NW
CH = 16
NCHUNK = PPW // CH
DF = L * F


def _encode_sc(coords_t, tables_flat):
    mesh = plsc.VectorSubcoreMesh(core_axis_name="c", subcore_axis_name="s")

    @functools.partial(
        pl.kernel,
        mesh=mesh,
        out_type=jax.ShapeDtypeStruct((DF, B), jnp.float32),
        scratch_types=[
            pltpu.VMEM((PPW,), jnp.float32),
            pltpu.VMEM((PPW,), jnp.float32),
            pltpu.VMEM((PPW,), jnp.float32),
            pltpu.VMEM((PPW,), jnp.float32),
            pltpu.VMEM((8 * CH,), jnp.int32),
            pltpu.VMEM((8 * CH,), jnp.int32),
            pltpu.VMEM((8 * CH,), jnp.int32),
            pltpu.VMEM((8 * CH,), jnp.int32),
            pltpu.VMEM((8 * CH,), jnp.float32),
            pltpu.VMEM((8 * CH,), jnp.float32),
            pltpu.VMEM((8 * CH,), jnp.float32),
            pltpu.VMEM((8 * CH,), jnp.float32),
            pltpu.VMEM((DF, PPW), jnp.float32),
            pltpu.SemaphoreType.DMA,
        ],
    )
    def enc(coords_hbm, tables_hbm, out_hbm,
            c0_v, c1_v, c2_v, c3_v,
            i0a_v, i0b_v, i1a_v, i1b_v,
            d0a_v, d0b_v, d1a_v, d1b_v,
            out_v, sem):
        wid = lax.axis_index("s") * 2 + lax.axis_index("c")
        base = wid * PPW
        cvs = (c0_v, c1_v, c2_v, c3_v)
        for d in range(4):
            pltpu.sync_copy(coords_hbm.at[d, pl.ds(base, PPW)], cvs[d])

        lanes = lax.iota(jnp.int32, CH)
        mask = jnp.uint32(T - 1)

        for l in range(L):
            scale = float(np.floor(BASE_RES * PER_LEVEL_SCALE ** l))
            lbase2 = 2 * l * T

            def body(g, carry, scale=scale, lbase2=lbase2, l=l):
                p0 = g * CH
                fr, om, av, bv = [], [], [], []
                for d in range(4):
                    x = cvs[d][pl.ds(p0, CH)] * scale
                    x0 = x.astype(jnp.int32)
                    f = x - x0.astype(jnp.float32)
                    fr.append(f)
                    om.append(1.0 - f)
                    xu = plsc.bitcast(x0, jnp.uint32)
                    p = jnp.uint32(PRIMES[d])
                    av.append(xu * p if d > 0 else xu)
                    bv.append((xu + jnp.uint32(1)) * p if d > 0
                              else xu + jnp.uint32(1))
                for c in range(16):
                    h = bv[0] if (c & 1) else av[0]
                    for d in range(1, 4):
                        h = h ^ (bv[d] if ((c >> d) & 1) else av[d])
                    e0 = (plsc.bitcast(h & mask, jnp.int32) * 2) + lbase2
                    sl = pl.ds((c & 7) * CH, CH)
                    if c < 8:
                        i0a_v[sl] = e0
                        i1a_v[sl] = e0 + 1
                    else:
                        i0b_v[sl] = e0
                        i1b_v[sl] = e0 + 1
                cps = [
                    pltpu.async_copy(tables_hbm.at[i0a_v], d0a_v, sem),
                    pltpu.async_copy(tables_hbm.at[i0b_v], d0b_v, sem),
                    pltpu.async_copy(tables_hbm.at[i1a_v], d1a_v, sem),
                    pltpu.async_copy(tables_hbm.at[i1b_v], d1b_v, sem),
                ]
                w01 = [om[0] * om[1], fr[0] * om[1], om[0] * fr[1],
                       fr[0] * fr[1]]
                w23 = [om[2] * om[3], fr[2] * om[3], om[2] * fr[3],
                       fr[2] * fr[3]]
                for cp in cps:
                    cp.wait()
                acc0 = jnp.zeros((CH,), jnp.float32)
                acc1 = jnp.zeros((CH,), jnp.float32)
                for c in range(16):
                    w = w01[c & 3] * w23[(c >> 2) & 3]
                    sl = pl.ds((c & 7) * CH, CH)
                    if c < 8:
                        f0, f1 = d0a_v[sl], d1a_v[sl]
                    else:
                        f0, f1 = d0b_v[sl], d1b_v[sl]
                    acc0 = acc0 + w * f0
                    acc1 = acc1 + w * f1
                out_v[2 * l, pl.ds(p0, CH)] = acc0
                out_v[2 * l + 1, pl.ds(p0, CH)] = acc1
                return carry

            lax.fori_loop(0, NCHUNK, body, 0)

        pltpu.sync_copy(out_v, out_hbm.at[:, pl.ds(base, PPW)])

    return enc(coords_t, tables_flat)


BS = 2048


def _mlp_body(feat_ref, w1_ref, b1_ref, w2_ref, b2_ref, w3_ref, b3_ref,
              out_ref):
    h = jnp.dot(feat_ref[...], w1_ref[...],
                preferred_element_type=jnp.float32) + b1_ref[...]
    h = jnp.maximum(h, 0.0)
    h = jnp.dot(h, w2_ref[...], preferred_element_type=jnp.float32) + b2_ref[...]
    h = jnp.maximum(h, 0.0)
    o = jnp.dot(h, w3_ref[...], preferred_element_type=jnp.float32) + b3_ref[...]
    out_ref[...] = jax.nn.sigmoid(o)


def _mlp_head(feat, W1, b1, W2, b2, W3, b3):
    d = DF
    W3p = jnp.zeros((128, 128), jnp.float32).at[:, :3].set(W3)
    b3p = jnp.zeros((1, 128), jnp.float32).at[:, :3].set(b3)
    out = pl.pallas_call(
        _mlp_body,
        grid=(B // BS,),
        in_specs=[
            pl.BlockSpec((BS, d), lambda i: (i, 0)),
            pl.BlockSpec((d, 256), lambda i: (0, 0)),
            pl.BlockSpec((1, 256), lambda i: (0, 0)),
            pl.BlockSpec((256, 128), lambda i: (0, 0)),
            pl.BlockSpec((1, 128), lambda i: (0, 0)),
            pl.BlockSpec((128, 128), lambda i: (0, 0)),
            pl.BlockSpec((1, 128), lambda i: (0, 0)),
        ],
        out_specs=pl.BlockSpec((BS, 128), lambda i: (i, 0)),
        out_shape=jax.ShapeDtypeStruct((B, 128), jnp.float32),
    )(feat, W1, b1.reshape(1, 256), W2, b2.reshape(1, 128), W3p, b3p)
    return out[:, :3]


def kernel(coords, tables, W1, b1, W2, b2, W3, b3):
    coords_t = coords.T
    tables_flat = tables.reshape(L * T * F)
    feat = _encode_sc(coords_t, tables_flat).T
    return _mlp_head(feat, W1, b1, W2, b2, W3, b3)


# speedup vs baseline: 1.0279x; 1.0279x over previous
import functools

import jax
import jax.numpy as jnp
import numpy as np
from jax import lax
from jax.experimental import pallas as pl
from jax.experimental.pallas import tpu as pltpu
from jax.experimental.pallas import tpu_sc as plsc

L = 16
T = 2 ** 19
F = 2
BASE_RES = 16
PER_LEVEL_SCALE = 1.5
PRIMES = (1, 2654435761, 805459861, 3674653429)

B = 16384
NW = 32
PPW = B // NW
CH = 16
NCHUNK = PPW // CH
DF = L * F
CPC = 16 * CH
ENT = PPW * 16 * F
SW = 128
NSTR = ENT // SW


def _encode_sc(coords_t, tables_flat):
    mesh = plsc.VectorSubcoreMesh(core_axis_name="c", subcore_axis_name="s")

    @functools.partial(
        pl.kernel,
        mesh=mesh,
        out_type=jax.ShapeDtypeStruct((DF, B), jnp.float32),
        scratch_types=[
            pltpu.VMEM((PPW,), jnp.float32),
            pltpu.VMEM((PPW,), jnp.float32),
            pltpu.VMEM((PPW,), jnp.float32),
            pltpu.VMEM((PPW,), jnp.float32),
            pltpu.VMEM((ENT,), jnp.int32),
            pltpu.VMEM((ENT,), jnp.float32),
            pltpu.VMEM((DF, PPW), jnp.float32),
            pltpu.SemaphoreType.DMA,
        ],
    )
    def enc(coords_hbm, tables_hbm, out_hbm,
            c0_v, c1_v, c2_v, c3_v, idx_v, data_v, out_v, sem):
        wid = lax.axis_index("s") * 2 + lax.axis_index("c")
        base = wid * PPW
        cvs = (c0_v, c1_v, c2_v, c3_v)
        for d in range(4):
            pltpu.sync_copy(coords_hbm.at[d, pl.ds(base, PPW)], cvs[d])

        lanes = lax.iota(jnp.int32, CH)
        mask = jnp.uint32(T - 1)

        def fracs(p0, scale):
            fr, om, xu = [], [], []
            for d in range(4):
                x = cvs[d][pl.ds(p0, CH)] * scale
                x0 = x.astype(jnp.int32)
                f = x - x0.astype(jnp.float32)
                fr.append(f)
                om.append(1.0 - f)
                xu.append(plsc.bitcast(x0, jnp.uint32))
            return fr, om, xu

        for l in range(L):
            scale = float(np.floor(BASE_RES * PER_LEVEL_SCALE ** l))
            lbase2 = 2 * l * T

            def abody(g, carry, scale=scale, lbase2=lbase2):
                p0 = g * CH
                _, _, xu = fracs(p0, scale)
                av, bv = [], []
                for d in range(4):
                    p = jnp.uint32(PRIMES[d])
                    av.append(xu[d] * p if d > 0 else xu[d])
                    bv.append((xu[d] + jnp.uint32(1)) * p if d > 0
                              else xu[d] + jnp.uint32(1))
                gb = g * (2 * CPC)
                for c in range(16):
                    h = bv[0] if (c & 1) else av[0]
                    for d in range(1, 4):
                        h = h ^ (bv[d] if ((c >> d) & 1) else av[d])
                    e0 = (plsc.bitcast(h & mask, jnp.int32) * 2) + lbase2
                    idx_v[pl.ds(gb + c * CH, CH)] = e0
                    idx_v[pl.ds(gb + CPC + c * CH, CH)] = e0 + 1
                return carry

            lax.fori_loop(0, NCHUNK, abody, 0)

            def bbody(j, carry):
                o = j * SW
                pltpu.async_copy(
                    tables_hbm.at[idx_v.at[pl.ds(o, SW)]],
                    data_v.at[pl.ds(o, SW)], sem)
                return carry

            lax.fori_loop(0, NSTR, bbody, 0)

            pltpu.make_async_copy(
                tables_hbm.at[pl.ds(0, ENT)], data_v, sem).wait()

            def dbody(g, carry, scale=scale, l=l):
                p0 = g * CH
                fr, om, _ = fracs(p0, scale)
                w01 = [om[0] * om[1], fr[0] * om[1], om[0] * fr[1],
                       fr[0] * fr[1]]
                w23 = [om[2] * om[3], fr[2] * om[3], om[2] * fr[3],
                       fr[2] * fr[3]]
                gb = g * (2 * CPC)
                acc0 = jnp.zeros((CH,), jnp.float32)
                acc1 = jnp.zeros((CH,), jnp.float32)
                for c in range(16):
                    w = w01[c & 3] * w23[(c >> 2) & 3]
                    f0 = data_v[pl.ds(gb + c * CH, CH)]
                    f1 = data_v[pl.ds(gb + CPC + c * CH, CH)]
                    acc0 = acc0 + w * f0
                    acc1 = acc1 + w * f1
                out_v[2 * l, pl.ds(p0, CH)] = acc0
                out_v[2 * l + 1, pl.ds(p0, CH)] = acc1
                return carry

            lax.fori_loop(0, NCHUNK, dbody, 0)

        pltpu.sync_copy(out_v, out_hbm.at[:, pl.ds(base, PPW)])

    return enc(coords_t, tables_flat)


BS = 2048


def _mlp_body(feat_ref, w1_ref, b1_ref, w2_ref, b2_ref, w3_ref, b3_ref,
              out_ref):
    h = jnp.dot(feat_ref[...], w1_ref[...],
                preferred_element_type=jnp.float32) + b1_ref[...]
    h = jnp.maximum(h, 0.0)
    h = jnp.dot(h, w2_ref[...], preferred_element_type=jnp.float32) + b2_ref[...]
    h = jnp.maximum(h, 0.0)
    o = jnp.dot(h, w3_ref[...], preferred_element_type=jnp.float32) + b3_ref[...]
    out_ref[...] = jax.nn.sigmoid(o)


def _mlp_head(feat, W1, b1, W2, b2, W3, b3):
    d = DF
    W3p = jnp.zeros((128, 128), jnp.float32).at[:, :3].set(W3)
    b3p = jnp.zeros((1, 128), jnp.float32).at[:, :3].set(b3)
    out = pl.pallas_call(
        _mlp_body,
        grid=(B // BS,),
        in_specs=[
            pl.BlockSpec((BS, d), lambda i: (i, 0)),
            pl.BlockSpec((d, 256), lambda i: (0, 0)),
            pl.BlockSpec((1, 256), lambda i: (0, 0)),
            pl.BlockSpec((256, 128), lambda i: (0, 0)),
            pl.BlockSpec((1, 128), lambda i: (0, 0)),
            pl.BlockSpec((128, 128), lambda i: (0, 0)),
            pl.BlockSpec((1, 128), lambda i: (0, 0)),
        ],
        out_specs=pl.BlockSpec((BS, 128), lambda i: (i, 0)),
        out_shape=jax.ShapeDtypeStruct((B, 128), jnp.float32),
    )(feat, W1, b1.reshape(1, 256), W2, b2.reshape(1, 128), W3p, b3p)
    return out[:, :3]


def kernel(coords, tables, W1, b1, W2, b2, W3, b3):
    coords_t = coords.T
    tables_flat = tables.reshape(L * T * F)
    feat = _encode_sc(coords_t, tables_flat).T
    return _mlp_head(feat, W1, b1, W2, b2, W3, b3)


# speedup vs baseline: 28.8901x; 28.1053x over previous
import functools

import jax
import jax.numpy as jnp
import numpy as np
from jax import lax
from jax.experimental import pallas as pl
from jax.experimental.pallas import tpu as pltpu
from jax.experimental.pallas import tpu_sc as plsc

L = 16
T = 2 ** 19
F = 2
BASE_RES = 16
PER_LEVEL_SCALE = 1.5
PRIMES = (1, 2654435761, 805459861, 3674653429)

B = 16384
NW = 32
PPW = B // NW
CH = 16
NCHUNK = PPW // CH
DF = L * F
CPC = 16 * CH
ENT = PPW * 16 * F
SW = 128
NSTR = ENT // SW


def _encode_sc(coords_t, tables_flat):
    mesh = plsc.VectorSubcoreMesh(core_axis_name="c", subcore_axis_name="s")

    @functools.partial(
        pl.kernel,
        mesh=mesh,
        out_type=jax.ShapeDtypeStruct((DF, B), jnp.float32),
        scratch_types=[
            pltpu.VMEM((PPW,), jnp.float32),
            pltpu.VMEM((PPW,), jnp.float32),
            pltpu.VMEM((PPW,), jnp.float32),
            pltpu.VMEM((PPW,), jnp.float32),
            pltpu.VMEM((ENT,), jnp.int32),
            pltpu.VMEM((ENT,), jnp.float32),
            pltpu.VMEM((DF, PPW), jnp.float32),
            pltpu.SemaphoreType.DMA,
        ],
    )
    def enc(coords_hbm, tables_hbm, out_hbm,
            c0_v, c1_v, c2_v, c3_v, idx_v, data_v, out_v, sem):
        wid = lax.axis_index("s") * 2 + lax.axis_index("c")
        base = wid * PPW
        cvs = (c0_v, c1_v, c2_v, c3_v)
        for d in range(4):
            pltpu.sync_copy(coords_hbm.at[d, pl.ds(base, PPW)], cvs[d])

        lanes = lax.iota(jnp.int32, CH)
        mask = jnp.uint32(T - 1)

        def fracs(p0, scale):
            fr, om, xu = [], [], []
            for d in range(4):
                x = cvs[d][pl.ds(p0, CH)] * scale
                x0 = x.astype(jnp.int32)
                f = x - x0.astype(jnp.float32)
                fr.append(f)
                om.append(1.0 - f)
                xu.append(plsc.bitcast(x0, jnp.uint32))
            return fr, om, xu

        for l in range(L):
            scale = float(np.floor(BASE_RES * PER_LEVEL_SCALE ** l))
            lbase2 = 2 * l * T

            def abody(g, carry, scale=scale, lbase2=lbase2):
                p0 = g * CH
                _, _, xu = fracs(p0, scale)
                av, bv = [], []
                for d in range(4):
                    p = jnp.uint32(PRIMES[d])
                    av.append(xu[d] * p if d > 0 else xu[d])
                    bv.append((xu[d] + jnp.uint32(1)) * p if d > 0
                              else xu[d] + jnp.uint32(1))
                gb = g * (2 * CPC)
                for c in range(16):
                    h = bv[0] if (c & 1) else av[0]
                    for d in range(1, 4):
                        h = h ^ (bv[d] if ((c >> d) & 1) else av[d])
                    hm = plsc.bitcast(h & mask, jnp.int32)
                    e0 = ((hm & ~127) * 2) + (hm & 127) + lbase2
                    idx_v[pl.ds(gb + c * CH, CH)] = e0
                    idx_v[pl.ds(gb + CPC + c * CH, CH)] = e0 + 128
                return carry

            lax.fori_loop(0, NCHUNK, abody, 0)

            def bbody(j, carry):
                o = j * SW
                pltpu.async_copy(
                    tables_hbm.at[idx_v.at[pl.ds(o, SW)]],
                    data_v.at[pl.ds(o, SW)], sem)
                return carry

            lax.fori_loop(0, NSTR, bbody, 0)

            pltpu.make_async_copy(
                tables_hbm.at[pl.ds(0, ENT)], data_v, sem).wait()

            def dbody(g, carry, scale=scale, l=l):
                p0 = g * CH
                fr, om, _ = fracs(p0, scale)
                w01 = [om[0] * om[1], fr[0] * om[1], om[0] * fr[1],
                       fr[0] * fr[1]]
                w23 = [om[2] * om[3], fr[2] * om[3], om[2] * fr[3],
                       fr[2] * fr[3]]
                gb = g * (2 * CPC)
                acc0 = jnp.zeros((CH,), jnp.float32)
                acc1 = jnp.zeros((CH,), jnp.float32)
                for c in range(16):
                    w = w01[c & 3] * w23[(c >> 2) & 3]
                    f0 = data_v[pl.ds(gb + c * CH, CH)]
                    f1 = data_v[pl.ds(gb + CPC + c * CH, CH)]
                    acc0 = acc0 + w * f0
                    acc1 = acc1 + w * f1
                out_v[2 * l, pl.ds(p0, CH)] = acc0
                out_v[2 * l + 1, pl.ds(p0, CH)] = acc1
                return carry

            lax.fori_loop(0, NCHUNK, dbody, 0)

        pltpu.sync_copy(out_v, out_hbm.at[:, pl.ds(base, PPW)])

    return enc(coords_t, tables_flat)


BS = 2048


def _mlp_body(feat_ref, w1_ref, b1_ref, w2_ref, b2_ref, w3_ref, b3_ref,
              out_ref):
    h = jnp.dot(feat_ref[...], w1_ref[...],
                preferred_element_type=jnp.float32) + b1_ref[...]
    h = jnp.maximum(h, 0.0)
    h = jnp.dot(h, w2_ref[...], preferred_element_type=jnp.float32) + b2_ref[...]
    h = jnp.maximum(h, 0.0)
    o = jnp.dot(h, w3_ref[...], preferred_element_type=jnp.float32) + b3_ref[...]
    out_ref[...] = jax.nn.sigmoid(o)


def _mlp_head(feat, W1, b1, W2, b2, W3, b3):
    d = DF
    W3p = jnp.zeros((128, 128), jnp.float32).at[:, :3].set(W3)
    b3p = jnp.zeros((1, 128), jnp.float32).at[:, :3].set(b3)
    out = pl.pallas_call(
        _mlp_body,
        grid=(B // BS,),
        in_specs=[
            pl.BlockSpec((BS, d), lambda i: (i, 0)),
            pl.BlockSpec((d, 256), lambda i: (0, 0)),
            pl.BlockSpec((1, 256), lambda i: (0, 0)),
            pl.BlockSpec((256, 128), lambda i: (0, 0)),
            pl.BlockSpec((1, 128), lambda i: (0, 0)),
            pl.BlockSpec((128, 128), lambda i: (0, 0)),
            pl.BlockSpec((1, 128), lambda i: (0, 0)),
        ],
        out_specs=pl.BlockSpec((BS, 128), lambda i: (i, 0)),
        out_shape=jax.ShapeDtypeStruct((B, 128), jnp.float32),
    )(feat, W1, b1.reshape(1, 256), W2, b2.reshape(1, 128), W3p, b3p)
    return out[:, :3]


def kernel(coords, tables, W1, b1, W2, b2, W3, b3):
    coords_t = coords.T
    tables_flat = tables.reshape(L, T // 128, 128, F).swapaxes(2, 3).reshape(-1)
    feat = _encode_sc(coords_t, tables_flat).T
    return _mlp_head(feat, W1, b1, W2, b2, W3, b3)


# speedup vs baseline: 32.1234x; 1.1119x over previous
import functools

import jax
import jax.numpy as jnp
import numpy as np
from jax import lax
from jax.experimental import pallas as pl
from jax.experimental.pallas import tpu as pltpu
from jax.experimental.pallas import tpu_sc as plsc

L = 16
T = 2 ** 19
F = 2
BASE_RES = 16
PER_LEVEL_SCALE = 1.5
PRIMES = (1, 2654435761, 805459861, 3674653429)

B = 16384
NW = 32
PPW = B // NW
CH = 16
NCHUNK = PPW // CH
DF = L * F
CPC = 16 * CH
ENT = PPW * 16 * F
SW = 128
NSTR = ENT // SW


def _encode_sc(coords_t, tables_flat):
    mesh = plsc.VectorSubcoreMesh(core_axis_name="c", subcore_axis_name="s")

    @functools.partial(
        pl.kernel,
        mesh=mesh,
        out_type=jax.ShapeDtypeStruct((DF, B), jnp.float32),
        scratch_types=[
            pltpu.VMEM((PPW,), jnp.float32),
            pltpu.VMEM((PPW,), jnp.float32),
            pltpu.VMEM((PPW,), jnp.float32),
            pltpu.VMEM((PPW,), jnp.float32),
            pltpu.VMEM((ENT,), jnp.int32),
            pltpu.VMEM((ENT,), jnp.int32),
            pltpu.VMEM((ENT,), jnp.float32),
            pltpu.VMEM((ENT,), jnp.float32),
            pltpu.VMEM((DF, PPW), jnp.float32),
            pltpu.SemaphoreType.DMA,
            pltpu.SemaphoreType.DMA,
        ],
    )
    def enc(coords_hbm, tables_hbm, out_hbm,
            c0_v, c1_v, c2_v, c3_v, idx0_v, idx1_v, dat0_v, dat1_v,
            out_v, sem0, sem1):
        wid = lax.axis_index("s") * 2 + lax.axis_index("c")
        base = wid * PPW
        cvs = (c0_v, c1_v, c2_v, c3_v)
        for d in range(4):
            pltpu.sync_copy(coords_hbm.at[d, pl.ds(base, PPW)], cvs[d])

        lanes = lax.iota(jnp.int32, CH)
        mask = jnp.uint32(T - 1)

        def fracs(p0, scale):
            fr, om, xu = [], [], []
            for d in range(4):
                x = cvs[d][pl.ds(p0, CH)] * scale
                x0 = x.astype(jnp.int32)
                f = x - x0.astype(jnp.float32)
                fr.append(f)
                om.append(1.0 - f)
                xu.append(plsc.bitcast(x0, jnp.uint32))
            return fr, om, xu

        def phase_a(l, idxv):
            scale = float(np.floor(BASE_RES * PER_LEVEL_SCALE ** l))
            lbase2 = 2 * l * T

            def abody(g, carry):
                p0 = g * CH
                _, _, xu = fracs(p0, scale)
                av, bv = [], []
                for d in range(4):
                    p = jnp.uint32(PRIMES[d])
                    av.append(xu[d] * p if d > 0 else xu[d])
                    bv.append((xu[d] + jnp.uint32(1)) * p if d > 0
                              else xu[d] + jnp.uint32(1))
                gb = g * (2 * CPC)
                for c in range(16):
                    h = bv[0] if (c & 1) else av[0]
                    for d in range(1, 4):
                        h = h ^ (bv[d] if ((c >> d) & 1) else av[d])
                    hm = plsc.bitcast(h & mask, jnp.int32)
                    e0 = ((hm & ~127) * 2) + (hm & 127) + lbase2
                    idxv[pl.ds(gb + c * CH, CH)] = e0
                    idxv[pl.ds(gb + CPC + c * CH, CH)] = e0 + 128
                return carry

            lax.fori_loop(0, NCHUNK, abody, 0)

        def fire(idxv, datv, semx):
            def bbody(j, carry):
                o = j * SW
                pltpu.async_copy(
                    tables_hbm.at[idxv.at[pl.ds(o, SW)]],
                    datv.at[pl.ds(o, SW)], semx)
                return carry

            lax.fori_loop(0, NSTR, bbody, 0)

        def drain(datv, semx):
            pltpu.make_async_copy(
                tables_hbm.at[pl.ds(0, ENT)], datv, semx).wait()

        def phase_d(l, datv):
            scale = float(np.floor(BASE_RES * PER_LEVEL_SCALE ** l))

            def dbody(g, carry):
                p0 = g * CH
                fr, om, _ = fracs(p0, scale)
                w01 = [om[0] * om[1], fr[0] * om[1], om[0] * fr[1],
                       fr[0] * fr[1]]
                w23 = [om[2] * om[3], fr[2] * om[3], om[2] * fr[3],
                       fr[2] * fr[3]]
                gb = g * (2 * CPC)
                acc0 = jnp.zeros((CH,), jnp.float32)
                acc1 = jnp.zeros((CH,), jnp.float32)
                for c in range(16):
                    w = w01[c & 3] * w23[(c >> 2) & 3]
                    f0 = datv[pl.ds(gb + c * CH, CH)]
                    f1 = datv[pl.ds(gb + CPC + c * CH, CH)]
                    acc0 = acc0 + w * f0
                    acc1 = acc1 + w * f1
                out_v[2 * l, pl.ds(p0, CH)] = acc0
                out_v[2 * l + 1, pl.ds(p0, CH)] = acc1
                return carry

            lax.fori_loop(0, NCHUNK, dbody, 0)

        bufs = ((idx0_v, dat0_v, sem0), (idx1_v, dat1_v, sem1))
        phase_a(0, idx0_v)
        fire(idx0_v, dat0_v, sem0)
        for l in range(1, L):
            ia, da, sa = bufs[l & 1]
            ip, dp, sp = bufs[(l - 1) & 1]
            phase_a(l, ia)
            fire(ia, da, sa)
            drain(dp, sp)
            phase_d(l - 1, dp)
        ip, dp, sp = bufs[(L - 1) & 1]
        drain(dp, sp)
        phase_d(L - 1, dp)

        pltpu.sync_copy(out_v, out_hbm.at[:, pl.ds(base, PPW)])

    return enc(coords_t, tables_flat)


BS = 2048


def _mlp_body(feat_ref, w1_ref, b1_ref, w2_ref, b2_ref, w3_ref, b3_ref,
              out_ref):
    h = jnp.dot(feat_ref[...], w1_ref[...],
                preferred_element_type=jnp.float32) + b1_ref[...]
    h = jnp.maximum(h, 0.0)
    h = jnp.dot(h, w2_ref[...], preferred_element_type=jnp.float32) + b2_ref[...]
    h = jnp.maximum(h, 0.0)
    o = jnp.dot(h, w3_ref[...], preferred_element_type=jnp.float32) + b3_ref[...]
    out_ref[...] = jax.nn.sigmoid(o)


def _mlp_head(feat, W1, b1, W2, b2, W3, b3):
    d = DF
    W3p = jnp.zeros((128, 128), jnp.float32).at[:, :3].set(W3)
    b3p = jnp.zeros((1, 128), jnp.float32).at[:, :3].set(b3)
    out = pl.pallas_call(
        _mlp_body,
        grid=(B // BS,),
        in_specs=[
            pl.BlockSpec((BS, d), lambda i: (i, 0)),
            pl.BlockSpec((d, 256), lambda i: (0, 0)),
            pl.BlockSpec((1, 256), lambda i: (0, 0)),
            pl.BlockSpec((256, 128), lambda i: (0, 0)),
            pl.BlockSpec((1, 128), lambda i: (0, 0)),
            pl.BlockSpec((128, 128), lambda i: (0, 0)),
            pl.BlockSpec((1, 128), lambda i: (0, 0)),
        ],
        out_specs=pl.BlockSpec((BS, 128), lambda i: (i, 0)),
        out_shape=jax.ShapeDtypeStruct((B, 128), jnp.float32),
    )(feat, W1, b1.reshape(1, 256), W2, b2.reshape(1, 128), W3p, b3p)
    return out[:, :3]


def kernel(coords, tables, W1, b1, W2, b2, W3, b3):
    coords_t = coords.T
    tables_flat = tables.reshape(L, T // 128, 128, F).swapaxes(2, 3).reshape(-1)
    feat = _encode_sc(coords_t, tables_flat).T
    return _mlp_head(feat, W1, b1, W2, b2, W3, b3)


# speedup vs baseline: 32.2494x; 1.0039x over previous
import functools

import jax
import jax.numpy as jnp
import numpy as np
from jax import lax
from jax.experimental import pallas as pl
from jax.experimental.pallas import tpu as pltpu
from jax.experimental.pallas import tpu_sc as plsc

L = 16
T = 2 ** 19
F = 2
BASE_RES = 16
PER_LEVEL_SCALE = 1.5
PRIMES = (1, 2654435761, 805459861, 3674653429)

B = 16384
NW = 32
PPW = B // NW
CH = 16
NCHUNK = PPW // CH
DF = L * F
CPC = 16 * CH
ENT = PPW * 16 * F
SW = 512
NSTR = ENT // SW


def _encode_sc(coords_t, tables_flat):
    mesh = plsc.VectorSubcoreMesh(core_axis_name="c", subcore_axis_name="s")

    @functools.partial(
        pl.kernel,
        mesh=mesh,
        out_type=jax.ShapeDtypeStruct((DF, B), jnp.float32),
        scratch_types=[
            pltpu.VMEM((PPW,), jnp.float32),
            pltpu.VMEM((PPW,), jnp.float32),
            pltpu.VMEM((PPW,), jnp.float32),
            pltpu.VMEM((PPW,), jnp.float32),
            pltpu.VMEM((ENT,), jnp.int32),
            pltpu.VMEM((ENT,), jnp.int32),
            pltpu.VMEM((ENT,), jnp.float32),
            pltpu.VMEM((ENT,), jnp.float32),
            pltpu.VMEM((DF, PPW), jnp.float32),
            pltpu.SemaphoreType.DMA,
            pltpu.SemaphoreType.DMA,
        ],
    )
    def enc(coords_hbm, tables_hbm, out_hbm,
            c0_v, c1_v, c2_v, c3_v, idx0_v, idx1_v, dat0_v, dat1_v,
            out_v, sem0, sem1):
        wid = lax.axis_index("s") * 2 + lax.axis_index("c")
        base = wid * PPW
        cvs = (c0_v, c1_v, c2_v, c3_v)
        for d in range(4):
            pltpu.sync_copy(coords_hbm.at[d, pl.ds(base, PPW)], cvs[d])

        lanes = lax.iota(jnp.int32, CH)
        mask = jnp.uint32(T - 1)

        def fracs(p0, scale):
            fr, om, xu = [], [], []
            for d in range(4):
                x = cvs[d][pl.ds(p0, CH)] * scale
                x0 = x.astype(jnp.int32)
                f = x - x0.astype(jnp.float32)
                fr.append(f)
                om.append(1.0 - f)
                xu.append(plsc.bitcast(x0, jnp.uint32))
            return fr, om, xu

        def phase_a(l, idxv):
            scale = float(np.floor(BASE_RES * PER_LEVEL_SCALE ** l))
            lbase2 = 2 * l * T

            def abody(g, carry):
                p0 = g * CH
                _, _, xu = fracs(p0, scale)
                av, bv = [], []
                for d in range(4):
                    p = jnp.uint32(PRIMES[d])
                    av.append(xu[d] * p if d > 0 else xu[d])
                    bv.append((xu[d] + jnp.uint32(1)) * p if d > 0
                              else xu[d] + jnp.uint32(1))
                gb = g * (2 * CPC)
                for c in range(16):
                    h = bv[0] if (c & 1) else av[0]
                    for d in range(1, 4):
                        h = h ^ (bv[d] if ((c >> d) & 1) else av[d])
                    hm = plsc.bitcast(h & mask, jnp.int32)
                    e0 = ((hm & ~127) * 2) + (hm & 127) + lbase2
                    idxv[pl.ds(gb + c * CH, CH)] = e0
                    idxv[pl.ds(gb + CPC + c * CH, CH)] = e0 + 128
                return carry

            lax.fori_loop(0, NCHUNK, abody, 0)

        def fire(idxv, datv, semx):
            def bbody(j, carry):
                o = j * SW
                pltpu.async_copy(
                    tables_hbm.at[idxv.at[pl.ds(o, SW)]],
                    datv.at[pl.ds(o, SW)], semx)
                return carry

            lax.fori_loop(0, NSTR, bbody, 0)

        def drain(datv, semx):
            pltpu.make_async_copy(
                tables_hbm.at[pl.ds(0, ENT)], datv, semx).wait()

        def phase_d(l, datv):
            scale = float(np.floor(BASE_RES * PER_LEVEL_SCALE ** l))

            def dbody(g, carry):
                p0 = g * CH
                fr, om, _ = fracs(p0, scale)
                w01 = [om[0] * om[1], fr[0] * om[1], om[0] * fr[1],
                       fr[0] * fr[1]]
                w23 = [om[2] * om[3], fr[2] * om[3], om[2] * fr[3],
                       fr[2] * fr[3]]
                gb = g * (2 * CPC)
                acc0 = jnp.zeros((CH,), jnp.float32)
                acc1 = jnp.zeros((CH,), jnp.float32)
                for c in range(16):
                    w = w01[c & 3] * w23[(c >> 2) & 3]
                    f0 = datv[pl.ds(gb + c * CH, CH)]
                    f1 = datv[pl.ds(gb + CPC + c * CH, CH)]
                    acc0 = acc0 + w * f0
                    acc1 = acc1 + w * f1
                out_v[2 * l, pl.ds(p0, CH)] = acc0
                out_v[2 * l + 1, pl.ds(p0, CH)] = acc1
                return carry

            lax.fori_loop(0, NCHUNK, dbody, 0)

        bufs = ((idx0_v, dat0_v, sem0), (idx1_v, dat1_v, sem1))
        phase_a(0, idx0_v)
        fire(idx0_v, dat0_v, sem0)
        for l in range(1, L):
            ia, da, sa = bufs[l & 1]
            ip, dp, sp = bufs[(l - 1) & 1]
            phase_a(l, ia)
            fire(ia, da, sa)
            drain(dp, sp)
            phase_d(l - 1, dp)
        ip, dp, sp = bufs[(L - 1) & 1]
        drain(dp, sp)
        phase_d(L - 1, dp)

        pltpu.sync_copy(out_v, out_hbm.at[:, pl.ds(base, PPW)])

    return enc(coords_t, tables_flat)


BS = 2048


def _mlp_body(feat_ref, w1_ref, b1_ref, w2_ref, b2_ref, w3_ref, b3_ref,
              out_ref):
    h = jnp.dot(feat_ref[...], w1_ref[...],
                preferred_element_type=jnp.float32) + b1_ref[...]
    h = jnp.maximum(h, 0.0)
    h = jnp.dot(h, w2_ref[...], preferred_element_type=jnp.float32) + b2_ref[...]
    h = jnp.maximum(h, 0.0)
    o = jnp.dot(h, w3_ref[...], preferred_element_type=jnp.float32) + b3_ref[...]
    out_ref[...] = jax.nn.sigmoid(o)


def _mlp_head(feat, W1, b1, W2, b2, W3, b3):
    d = DF
    W3p = jnp.zeros((128, 128), jnp.float32).at[:, :3].set(W3)
    b3p = jnp.zeros((1, 128), jnp.float32).at[:, :3].set(b3)
    out = pl.pallas_call(
        _mlp_body,
        grid=(B // BS,),
        in_specs=[
            pl.BlockSpec((BS, d), lambda i: (i, 0)),
            pl.BlockSpec((d, 256), lambda i: (0, 0)),
            pl.BlockSpec((1, 256), lambda i: (0, 0)),
            pl.BlockSpec((256, 128), lambda i: (0, 0)),
            pl.BlockSpec((1, 128), lambda i: (0, 0)),
            pl.BlockSpec((128, 128), lambda i: (0, 0)),
            pl.BlockSpec((1, 128), lambda i: (0, 0)),
        ],
        out_specs=pl.BlockSpec((BS, 128), lambda i: (i, 0)),
        out_shape=jax.ShapeDtypeStruct((B, 128), jnp.float32),
    )(feat, W1, b1.reshape(1, 256), W2, b2.reshape(1, 128), W3p, b3p)
    return out[:, :3]


def kernel(coords, tables, W1, b1, W2, b2, W3, b3):
    coords_t = coords.T
    tables_flat = tables.reshape(L, T // 128, 128, F).swapaxes(2, 3).reshape(-1)
    feat = _encode_sc(coords_t, tables_flat).T
    return _mlp_head(feat, W1, b1, W2, b2, W3, b3)


# speedup vs baseline: 33.5366x; 1.0399x over previous
import functools

import jax
import jax.numpy as jnp
import numpy as np
from jax import lax
from jax.experimental import pallas as pl
from jax.experimental.pallas import tpu as pltpu
from jax.experimental.pallas import tpu_sc as plsc

L = 16
T = 2 ** 19
F = 2
BASE_RES = 16
PER_LEVEL_SCALE = 1.5
PRIMES = (1, 2654435761, 805459861, 3674653429)

B = 16384
NW = 32
PPW = B // NW
CH = 16
NCHUNK = PPW // CH
DF = L * F
CPC = 16 * CH
ENT = PPW * 16 * F
SW = 512
NSTR = ENT // SW


def _encode_sc(coords_t, tables_flat):
    mesh = plsc.VectorSubcoreMesh(core_axis_name="c", subcore_axis_name="s")

    @functools.partial(
        pl.kernel,
        mesh=mesh,
        out_type=jax.ShapeDtypeStruct((DF, B), jnp.float32),
        scratch_types=[
            pltpu.VMEM((PPW,), jnp.float32),
            pltpu.VMEM((PPW,), jnp.float32),
            pltpu.VMEM((PPW,), jnp.float32),
            pltpu.VMEM((PPW,), jnp.float32),
            pltpu.VMEM((ENT,), jnp.int32),
            pltpu.VMEM((ENT,), jnp.int32),
            pltpu.VMEM((ENT,), jnp.float32),
            pltpu.VMEM((ENT,), jnp.float32),
            pltpu.VMEM((DF, PPW), jnp.float32),
            pltpu.SemaphoreType.DMA,
            pltpu.SemaphoreType.DMA,
        ],
    )
    def enc(coords_hbm, tables_hbm, out_hbm,
            c0_v, c1_v, c2_v, c3_v, idx0_v, idx1_v, dat0_v, dat1_v,
            out_v, sem0, sem1):
        wid = lax.axis_index("s") * 2 + lax.axis_index("c")
        base = wid * PPW
        cvs = (c0_v, c1_v, c2_v, c3_v)
        for d in range(4):
            pltpu.sync_copy(coords_hbm.at[d, pl.ds(base, PPW)], cvs[d])

        lanes = lax.iota(jnp.int32, CH)
        mask = jnp.uint32(T - 1)

        def fracs(p0, scale):
            fr, om, xu = [], [], []
            for d in range(4):
                x = cvs[d][pl.ds(p0, CH)] * scale
                x0 = x.astype(jnp.int32)
                f = x - x0.astype(jnp.float32)
                fr.append(f)
                om.append(1.0 - f)
                xu.append(plsc.bitcast(x0, jnp.uint32))
            return fr, om, xu

        def phase_a(l, idxv):
            scale = float(np.floor(BASE_RES * PER_LEVEL_SCALE ** l))
            lbase2 = 2 * l * T

            def abody(g, carry):
                p0 = g * CH
                _, _, xu = fracs(p0, scale)
                av, bv = [], []
                for d in range(4):
                    p = jnp.uint32(PRIMES[d])
                    av.append(xu[d] * p if d > 0 else xu[d])
                    bv.append((xu[d] + jnp.uint32(1)) * p if d > 0
                              else xu[d] + jnp.uint32(1))
                gb = g * (2 * CPC)
                for c in range(16):
                    h = bv[0] if (c & 1) else av[0]
                    for d in range(1, 4):
                        h = h ^ (bv[d] if ((c >> d) & 1) else av[d])
                    hm = plsc.bitcast(h & mask, jnp.int32)
                    e0 = ((hm & ~127) * 2) + (hm & 127) + lbase2
                    idxv[pl.ds(gb + c * CH, CH)] = e0
                    idxv[pl.ds(gb + CPC + c * CH, CH)] = e0 + 128
                return carry

            lax.fori_loop(0, NCHUNK, abody, 0)

        def fire(idxv, datv, semx):
            def bbody(j, carry):
                o = j * SW
                pltpu.async_copy(
                    tables_hbm.at[idxv.at[pl.ds(o, SW)]],
                    datv.at[pl.ds(o, SW)], semx)
                return carry

            lax.fori_loop(0, NSTR, bbody, 0)

        def drain(datv, semx):
            pltpu.make_async_copy(
                tables_hbm.at[pl.ds(0, ENT)], datv, semx).wait()

        def phase_d(l, datv):
            scale = float(np.floor(BASE_RES * PER_LEVEL_SCALE ** l))

            def dbody(g, carry):
                p0 = g * CH
                fr, om, _ = fracs(p0, scale)
                w01 = [om[0] * om[1], fr[0] * om[1], om[0] * fr[1],
                       fr[0] * fr[1]]
                w23 = [om[2] * om[3], fr[2] * om[3], om[2] * fr[3],
                       fr[2] * fr[3]]
                gb = g * (2 * CPC)
                acc0 = jnp.zeros((CH,), jnp.float32)
                acc1 = jnp.zeros((CH,), jnp.float32)
                for c in range(16):
                    w = w01[c & 3] * w23[(c >> 2) & 3]
                    f0 = datv[pl.ds(gb + c * CH, CH)]
                    f1 = datv[pl.ds(gb + CPC + c * CH, CH)]
                    acc0 = acc0 + w * f0
                    acc1 = acc1 + w * f1
                out_v[2 * l, pl.ds(p0, CH)] = acc0
                out_v[2 * l + 1, pl.ds(p0, CH)] = acc1
                return carry

            lax.fori_loop(0, NCHUNK, dbody, 0)

        bufs = ((idx0_v, dat0_v, sem0), (idx1_v, dat1_v, sem1))
        phase_a(0, idx0_v)
        fire(idx0_v, dat0_v, sem0)
        for l in range(1, L):
            ia, da, sa = bufs[l & 1]
            ip, dp, sp = bufs[(l - 1) & 1]
            phase_a(l, ia)
            fire(ia, da, sa)
            drain(dp, sp)
            phase_d(l - 1, dp)
        ip, dp, sp = bufs[(L - 1) & 1]
        drain(dp, sp)
        phase_d(L - 1, dp)

        pltpu.sync_copy(out_v, out_hbm.at[:, pl.ds(base, PPW)])

    return enc(coords_t, tables_flat)


BS = 2048


def _mlp_body(featT_ref, w1t_ref, b1_ref, w2t_ref, b2_ref, w3t_ref, b3_ref,
              out_ref):
    x = featT_ref[...]
    h = jnp.dot(w1t_ref[...], x,
                preferred_element_type=jnp.float32) + b1_ref[...]
    h = jnp.maximum(h, 0.0)
    h = jnp.dot(w2t_ref[...], h,
                preferred_element_type=jnp.float32) + b2_ref[...]
    h = jnp.maximum(h, 0.0)
    o = jnp.dot(w3t_ref[...], h,
                preferred_element_type=jnp.float32) + b3_ref[...]
    out_ref[...] = jax.nn.sigmoid(o)


def _mlp_head(featT, W1, b1, W2, b2, W3, b3):
    d = DF
    W3tp = jnp.zeros((128, 128), jnp.float32).at[:3, :].set(W3.T)
    b3p = jnp.zeros((128, 1), jnp.float32).at[:3, :].set(b3.reshape(3, 1))
    out = pl.pallas_call(
        _mlp_body,
        grid=(B // BS,),
        in_specs=[
            pl.BlockSpec((d, BS), lambda i: (0, i)),
            pl.BlockSpec((256, d), lambda i: (0, 0)),
            pl.BlockSpec((256, 1), lambda i: (0, 0)),
            pl.BlockSpec((128, 256), lambda i: (0, 0)),
            pl.BlockSpec((128, 1), lambda i: (0, 0)),
            pl.BlockSpec((128, 128), lambda i: (0, 0)),
            pl.BlockSpec((128, 1), lambda i: (0, 0)),
        ],
        out_specs=pl.BlockSpec((128, BS), lambda i: (0, i)),
        out_shape=jax.ShapeDtypeStruct((128, B), jnp.float32),
    )(featT, W1.T, b1.reshape(256, 1), W2.T, b2.reshape(128, 1), W3tp, b3p)
    return out[:3, :].T


def kernel(coords, tables, W1, b1, W2, b2, W3, b3):
    coords_t = coords.T
    tables_flat = tables.reshape(L, T // 128, 128, F).swapaxes(2, 3).reshape(-1)
    featT = _encode_sc(coords_t, tables_flat)
    return _mlp_head(featT, W1, b1, W2, b2, W3, b3)
